# baseline (speedup 1.0000x reference)
import functools

import jax
import jax.numpy as jnp
from jax import lax
from jax.experimental import pallas as pl
from jax.experimental.pallas import tpu as pltpu
from jax.experimental.pallas import tpu_sc as plsc

B = 1024
C = 100000
CBLK = 2000
NBLK = C // CBLK // 2
TAU = 1.0

_NC = 2
_NS = 16
_L = 16
_NW = _NC * _NS
_BPW = B // _NW


def _sc_gather_body(y_hbm, occ_hbm, occy_out, y_v, occ_v, sem0):
    wid = lax.axis_index("s") * _NC + lax.axis_index("c")
    base = wid * _BPW
    pltpu.sync_copy(y_hbm.at[pl.ds(base, _BPW)], y_v)
    pltpu.async_copy(occ_hbm.at[y_v], occ_v, sem0).wait()
    pltpu.sync_copy(occ_v, occy_out.at[pl.ds(base, _BPW)])


@functools.cache
def _sc_gather():
    return pl.kernel(
        _sc_gather_body,
        mesh=plsc.VectorSubcoreMesh(core_axis_name="c", subcore_axis_name="s"),
        out_type=jax.ShapeDtypeStruct((B,), jnp.float32),
        scratch_types=[
            pltpu.VMEM((_BPW,), jnp.int32),
            pltpu.VMEM((_BPW,), jnp.float32),
            pltpu.SemaphoreType.DMA,
        ],
    )


def _colsum_body(y_ref, x1_ref, x2_ref, zt_ref, zy_ref, zt_acc, zy_acc):
    j = pl.program_id(0)

    @pl.when(j == 0)
    def _init():
        zt_acc[...] = jnp.zeros_like(zt_acc)
        zy_acc[...] = jnp.zeros_like(zy_acc)

    yb = y_ref[...]
    x1 = x1_ref[...]
    x2 = x2_ref[...]
    zt_acc[...] += (jnp.sum(x1, axis=0, keepdims=True)
                    + jnp.sum(x2, axis=0, keepdims=True))
    rows = j * CBLK + lax.broadcasted_iota(jnp.int32, (CBLK, B), 0)
    zy_acc[...] += (jnp.sum(jnp.where(rows == yb, x1, 0.0),
                            axis=0, keepdims=True)
                    + jnp.sum(jnp.where(rows + (C // 2) == yb, x2, 0.0),
                              axis=0, keepdims=True))

    @pl.when(j == NBLK - 1)
    def _emit():
        zt_ref[...] = zt_acc[...]
        zy_ref[...] = zy_acc[...]


def _colsum(y_row, logit_t):
    return pl.pallas_call(
        _colsum_body,
        grid=(NBLK,),
        in_specs=[
            pl.BlockSpec((1, B), lambda j: (0, 0)),
            pl.BlockSpec((CBLK, B), lambda j: (j, 0)),
            pl.BlockSpec((CBLK, B), lambda j: (j + NBLK, 0)),
        ],
        out_specs=[
            pl.BlockSpec((1, B), lambda j: (0, 0)),
            pl.BlockSpec((1, B), lambda j: (0, 0)),
        ],
        out_shape=[
            jax.ShapeDtypeStruct((1, B), jnp.float32),
            jax.ShapeDtypeStruct((1, B), jnp.float32),
        ],
        scratch_shapes=[
            pltpu.VMEM((1, B), jnp.float32),
            pltpu.VMEM((1, B), jnp.float32),
        ],
        compiler_params=pltpu.CompilerParams(
            dimension_semantics=("arbitrary",),
        ),
    )(y_row, logit_t, logit_t)


def _loss_body(zt_ref, zy_ref, occ_ref, loss_ref):
    z_total = zt_ref[...]
    z_y = zy_ref[...]
    occ_y = jnp.maximum(occ_ref[...], jnp.float32(1e-8))
    shift = TAU * lax.rsqrt(jnp.sqrt(occ_y))
    y_logit = jnp.exp(z_y - shift)
    den = y_logit + z_total - z_y
    safe = jnp.where(den == 0.0, jnp.float32(1.0), den)
    ratio = jnp.where(den == 0.0, jnp.float32(0.0), y_logit / safe)
    loss_ref[0, 0] = jnp.mean(-jnp.log(ratio))


def _loss(zt, zy, occ_row):
    return pl.pallas_call(
        _loss_body,
        out_specs=pl.BlockSpec(memory_space=pltpu.SMEM),
        out_shape=jax.ShapeDtypeStruct((1, 1), jnp.float32),
    )(zt, zy, occ_row)


def kernel(y, logit, class_occ):
    y_flat = jnp.reshape(y, (B,)).astype(jnp.int32)
    occ_y = _sc_gather()(y_flat, class_occ)
    zt, zy = _colsum(jnp.reshape(y_flat, (1, B)), logit.T)
    loss = _loss(zt, zy, jnp.reshape(occ_y, (1, B)))
    return jnp.reshape(loss, ())


# speedup vs baseline: 1.0003x; 1.0003x over previous
import functools

import jax
import jax.numpy as jnp
from jax import lax
from jax.experimental import pallas as pl
from jax.experimental.pallas import tpu as pltpu
from jax.experimental.pallas import tpu_sc as plsc

B = 1024
C = 100000
CBLK = 4000
NBLK = C // CBLK
TAU = 1.0

_NC = 2
_NS = 16
_L = 16
_NW = _NC * _NS
_BPW = B // _NW


def _sc_gather_body(y_hbm, occ_hbm, occy_out, y_v, occ_v, sem0):
    wid = lax.axis_index("s") * _NC + lax.axis_index("c")
    base = wid * _BPW
    pltpu.sync_copy(y_hbm.at[pl.ds(base, _BPW)], y_v)
    pltpu.async_copy(occ_hbm.at[y_v], occ_v, sem0).wait()
    pltpu.sync_copy(occ_v, occy_out.at[pl.ds(base, _BPW)])


@functools.cache
def _sc_gather():
    return pl.kernel(
        _sc_gather_body,
        mesh=plsc.VectorSubcoreMesh(core_axis_name="c", subcore_axis_name="s"),
        out_type=jax.ShapeDtypeStruct((B,), jnp.float32),
        scratch_types=[
            pltpu.VMEM((_BPW,), jnp.int32),
            pltpu.VMEM((_BPW,), jnp.float32),
            pltpu.SemaphoreType.DMA,
        ],
    )


def _colsum_body(y_ref, x_ref, zt_ref, zy_ref, zt_acc, zy_acc):
    j = pl.program_id(0)

    @pl.when(j == 0)
    def _init():
        zt_acc[...] = jnp.zeros_like(zt_acc)
        zy_acc[...] = jnp.zeros_like(zy_acc)

    x = x_ref[...]
    zt_acc[...] += jnp.sum(x, axis=0, keepdims=True)
    rows = j * CBLK + lax.broadcasted_iota(jnp.int32, (CBLK, B), 0)
    zy_acc[...] += jnp.sum(jnp.where(rows == y_ref[...], x, 0.0),
                           axis=0, keepdims=True)

    @pl.when(j == NBLK - 1)
    def _emit():
        zt_ref[...] = zt_acc[...]
        zy_ref[...] = zy_acc[...]


def _colsum(y_row, logit_t):
    return pl.pallas_call(
        _colsum_body,
        grid=(NBLK,),
        in_specs=[
            pl.BlockSpec((1, B), lambda j: (0, 0)),
            pl.BlockSpec((CBLK, B), lambda j: (j, 0)),
        ],
        out_specs=[
            pl.BlockSpec((1, B), lambda j: (0, 0)),
            pl.BlockSpec((1, B), lambda j: (0, 0)),
        ],
        out_shape=[
            jax.ShapeDtypeStruct((1, B), jnp.float32),
            jax.ShapeDtypeStruct((1, B), jnp.float32),
        ],
        scratch_shapes=[
            pltpu.VMEM((1, B), jnp.float32),
            pltpu.VMEM((1, B), jnp.float32),
        ],
        compiler_params=pltpu.CompilerParams(
            dimension_semantics=("arbitrary",),
        ),
    )(y_row, logit_t)


def _loss_body(zt_ref, zy_ref, occ_ref, loss_ref):
    z_total = zt_ref[...]
    z_y = zy_ref[...]
    occ_y = jnp.maximum(occ_ref[...], jnp.float32(1e-8))
    shift = TAU * lax.rsqrt(jnp.sqrt(occ_y))
    y_logit = jnp.exp(z_y - shift)
    den = y_logit + z_total - z_y
    safe = jnp.where(den == 0.0, jnp.float32(1.0), den)
    ratio = jnp.where(den == 0.0, jnp.float32(0.0), y_logit / safe)
    loss_ref[0, 0] = jnp.mean(-jnp.log(ratio))


def _loss(zt, zy, occ_row):
    return pl.pallas_call(
        _loss_body,
        out_specs=pl.BlockSpec(memory_space=pltpu.SMEM),
        out_shape=jax.ShapeDtypeStruct((1, 1), jnp.float32),
    )(zt, zy, occ_row)


def kernel(y, logit, class_occ):
    y_flat = jnp.reshape(y, (B,)).astype(jnp.int32)
    occ_y = _sc_gather()(y_flat, class_occ)
    zt, zy = _colsum(jnp.reshape(y_flat, (1, B)), logit.T)
    loss = _loss(zt, zy, jnp.reshape(occ_y, (1, B)))
    return jnp.reshape(loss, ())
